# baseline (device time: 34451 ns/iter reference)
import functools

import jax
import jax.numpy as jnp
from jax import lax
from jax.experimental import pallas as pl
from jax.experimental.pallas import tpu as pltpu

N_DEV = 4
N_SRC = 2
B = 2
SQ = 128
SKV_LOC = 128
H_LOC = 4
DH = 64
D_MODEL = 512
NEG = -1e9

_DeviceIdType = getattr(pl, "DeviceIdType", None) or pltpu.DeviceIdType
_sem_signal = getattr(pl, "semaphore_signal", None) or pltpu.semaphore_signal
_sem_wait = getattr(pl, "semaphore_wait", None) or pltpu.semaphore_wait
_CompilerParams = getattr(pltpu, "CompilerParams", None) or getattr(
    pltpu, "TPUCompilerParams"
)


def kernel(x, Wq, K_ext, V_ext, Wo):
    def body(x_ref, wq_ref, k_ref, v_ref, wo_ref, out_ref,
             kv_send, kv_recv, pout_send, pout_recv,
             kv_send_sems, kv_recv_sems, p_send_sems, p_recv_sems):
        my = lax.axis_index("i")

        barrier = pltpu.get_barrier_semaphore()
        for d in range(1, N_DEV):
            _sem_signal(barrier, inc=1, device_id=((my + d) % N_DEV,),
                        device_id_type=_DeviceIdType.MESH)
        _sem_wait(barrier, N_DEV - 1)

        def _kv_send_desc(p, t):
            return pltpu.make_async_remote_copy(
                src_ref=kv_send.at[t], dst_ref=kv_recv.at[p],
                send_sem=kv_send_sems.at[t], recv_sem=kv_recv_sems.at[p],
                device_id=(t,), device_id_type=_DeviceIdType.MESH)

        for p in range(N_SRC):
            @pl.when(my == p)
            def _(p=p):
                kv_recv[p, 0] = k_ref[:, :, 4 * p:4 * p + 4, :].astype(jnp.bfloat16)
                kv_recv[p, 1] = v_ref[:, :, 4 * p:4 * p + 4, :].astype(jnp.bfloat16)
                for t in range(N_DEV):
                    if t == p:
                        continue
                    kv_send[t, 0] = k_ref[:, :, 4 * t:4 * t + 4, :].astype(jnp.bfloat16)
                    kv_send[t, 1] = v_ref[:, :, 4 * t:4 * t + 4, :].astype(jnp.bfloat16)
                for t in range(N_DEV):
                    if t != p:
                        _kv_send_desc(p, t).start()

        wqb = wq_ref[...].astype(jnp.bfloat16)
        wob = wo_ref[...].astype(jnp.bfloat16)
        qs = []
        for b in range(B):
            qs.append(lax.dot(x_ref[b].astype(jnp.bfloat16), wqb,
                              preferred_element_type=jnp.float32))

        for p in range(N_SRC):
            @pl.when(my != p)
            def _(p=p):
                _kv_send_desc(p, p).wait_recv()

        iota_i = lax.broadcasted_iota(jnp.int32, (SQ, SKV_LOC), 0)
        iota_j = lax.broadcasted_iota(jnp.int32, (SQ, SKV_LOC), 1)
        causal = iota_j <= iota_i
        for b in range(B):
            k0 = kv_recv[0, 0, b]
            k1 = kv_recv[1, 0, b]
            v0 = kv_recv[0, 1, b]
            v1 = kv_recv[1, 1, b]
            ctx_heads = []
            for h in range(H_LOC):
                q_bh = qs[b][:, h * DH:(h + 1) * DH].astype(jnp.bfloat16)
                s0 = lax.dot_general(q_bh, k0[:, h, :], (((1,), (1,)), ((), ())),
                                     preferred_element_type=jnp.float32) * 0.125
                s1 = lax.dot_general(q_bh, k1[:, h, :], (((1,), (1,)), ((), ())),
                                     preferred_element_type=jnp.float32) * 0.125
                s1 = jnp.where(causal, s1, NEG)
                s = jnp.concatenate([s0, s1], axis=1)
                m = jnp.max(s, axis=1, keepdims=True)
                w = jnp.exp(s - m)
                w = (w / jnp.sum(w, axis=1, keepdims=True)).astype(jnp.bfloat16)
                c = (lax.dot(w[:, :SKV_LOC], v0[:, h, :],
                             preferred_element_type=jnp.float32)
                     + lax.dot(w[:, SKV_LOC:], v1[:, h, :],
                               preferred_element_type=jnp.float32))
                ctx_heads.append(c)
            ctx_b = jnp.concatenate(ctx_heads, axis=1).astype(jnp.bfloat16)
            pout_b = lax.dot(ctx_b, wob, preferred_element_type=jnp.float32)
            out_ref[b] = pout_b
            pout_send[b] = pout_b.astype(jnp.bfloat16)

        def _p_desc(slot, tgt):
            return pltpu.make_async_remote_copy(
                src_ref=pout_send, dst_ref=pout_recv.at[slot],
                send_sem=p_send_sems.at[slot], recv_sem=p_recv_sems.at[slot],
                device_id=(tgt,), device_id_type=_DeviceIdType.MESH)

        for d in range(1, N_DEV):
            _p_desc(d - 1, (my + d) % N_DEV).start()
        for r in range(N_DEV - 1):
            _p_desc(r, my).wait_recv()

        for b in range(B):
            out_ref[b] = (out_ref[b]
                          + pout_recv[0, b].astype(jnp.float32)
                          + pout_recv[1, b].astype(jnp.float32)
                          + pout_recv[2, b].astype(jnp.float32))

        for d in range(1, N_DEV):
            _p_desc(d - 1, (my + d) % N_DEV).wait_send()
        for p in range(N_SRC):
            @pl.when(my == p)
            def _(p=p):
                for t in range(N_DEV):
                    if t != p:
                        _kv_send_desc(p, t).wait_send()

        @functools.partial(pl.run_scoped, exit_sem=pltpu.SemaphoreType.REGULAR)
        def _(exit_sem):
            for d in range(1, N_DEV):
                _sem_signal(exit_sem, inc=1, device_id=((my + d) % N_DEV,),
                            device_id_type=_DeviceIdType.MESH)
            _sem_wait(exit_sem, N_DEV - 1)

    return pl.pallas_call(
        body,
        out_shape=jax.ShapeDtypeStruct((B, SQ, D_MODEL), jnp.float32),
        in_specs=[pl.BlockSpec(memory_space=pltpu.VMEM)] * 5,
        out_specs=pl.BlockSpec(memory_space=pltpu.VMEM),
        scratch_shapes=[
            pltpu.VMEM((N_DEV, 2, B, SKV_LOC, H_LOC, DH), jnp.bfloat16),
            pltpu.VMEM((N_SRC, 2, B, SKV_LOC, H_LOC, DH), jnp.bfloat16),
            pltpu.VMEM((B, SQ, D_MODEL), jnp.bfloat16),
            pltpu.VMEM((N_DEV - 1, B, SQ, D_MODEL), jnp.bfloat16),
            pltpu.SemaphoreType.DMA((N_DEV,)),
            pltpu.SemaphoreType.DMA((N_SRC,)),
            pltpu.SemaphoreType.DMA((N_DEV - 1,)),
            pltpu.SemaphoreType.DMA((N_DEV - 1,)),
        ],
        compiler_params=_CompilerParams(collective_id=0),
    )(x, Wq, K_ext, V_ext, Wo)


# device time: 27326 ns/iter; 1.2607x vs baseline; 1.2607x over previous
import functools

import jax
import jax.numpy as jnp
from jax import lax
from jax.experimental import pallas as pl
from jax.experimental.pallas import tpu as pltpu

N_DEV = 4
N_SRC = 2
B = 2
SQ = 128
SKV_LOC = 128
H_LOC = 4
DH = 64
D_MODEL = 512
NEG = -1e9

_DeviceIdType = getattr(pl, "DeviceIdType", None) or pltpu.DeviceIdType
_sem_signal = getattr(pl, "semaphore_signal", None) or pltpu.semaphore_signal
_sem_wait = getattr(pl, "semaphore_wait", None) or pltpu.semaphore_wait
_CompilerParams = getattr(pltpu, "CompilerParams", None) or getattr(
    pltpu, "TPUCompilerParams"
)


def kernel(x, Wq, K_ext, V_ext, Wo):
    K2 = K_ext.reshape(B, SKV_LOC, 16 * DH)
    V2 = V_ext.reshape(B, SKV_LOC, 16 * DH)
    HD = H_LOC * DH

    def body(x_ref, wq_ref, k_ref, v_ref, wo_ref, out_ref,
             kv_send, kv_recv, pout_send, pout_recv,
             kv_send_sems, kv_recv_sems, p_send_sems, p_recv_sems):
        my = lax.axis_index("i")

        barrier = pltpu.get_barrier_semaphore()
        for d in range(1, N_DEV):
            _sem_signal(barrier, inc=1, device_id=((my + d) % N_DEV,),
                        device_id_type=_DeviceIdType.MESH)
        _sem_wait(barrier, N_DEV - 1)

        def _kv_send_desc(p, t):
            return pltpu.make_async_remote_copy(
                src_ref=kv_send.at[t], dst_ref=kv_recv.at[p],
                send_sem=kv_send_sems.at[t], recv_sem=kv_recv_sems.at[p],
                device_id=(t,), device_id_type=_DeviceIdType.MESH)

        for p in range(N_SRC):
            @pl.when(my == p)
            def _(p=p):
                for t in range(N_DEV):
                    if t == p:
                        continue
                    kv_send[t, 0] = k_ref[:, :, HD * t:HD * (t + 1)].astype(jnp.bfloat16)
                    kv_send[t, 1] = v_ref[:, :, HD * t:HD * (t + 1)].astype(jnp.bfloat16)
                    _kv_send_desc(p, t).start()
                kv_recv[p, 0] = k_ref[:, :, HD * p:HD * (p + 1)].astype(jnp.bfloat16)
                kv_recv[p, 1] = v_ref[:, :, HD * p:HD * (p + 1)].astype(jnp.bfloat16)

        wqb = wq_ref[...].astype(jnp.bfloat16)
        wob = wo_ref[...].astype(jnp.bfloat16)
        qs = []
        for b in range(B):
            qs.append(lax.dot(x_ref[b].astype(jnp.bfloat16), wqb,
                              preferred_element_type=jnp.float32))

        for p in range(N_SRC):
            @pl.when(my != p)
            def _(p=p):
                _kv_send_desc(p, p).wait_recv()

        iota_i = lax.broadcasted_iota(jnp.int32, (SQ, SKV_LOC), 0)
        iota_j = lax.broadcasted_iota(jnp.int32, (SQ, SKV_LOC), 1)
        causal = iota_j <= iota_i
        for b in range(B):
            k0 = kv_recv[0, 0, b]
            k1 = kv_recv[1, 0, b]
            v0 = kv_recv[0, 1, b]
            v1 = kv_recv[1, 1, b]
            ctx_heads = []
            for h in range(H_LOC):
                hs = slice(h * DH, (h + 1) * DH)
                q_bh = qs[b][:, hs].astype(jnp.bfloat16)
                s0 = lax.dot_general(q_bh, k0[:, hs], (((1,), (1,)), ((), ())),
                                     preferred_element_type=jnp.float32) * 0.125
                s1 = lax.dot_general(q_bh, k1[:, hs], (((1,), (1,)), ((), ())),
                                     preferred_element_type=jnp.float32) * 0.125
                s1 = jnp.where(causal, s1, NEG)
                s = jnp.concatenate([s0, s1], axis=1)
                m = jnp.max(s, axis=1, keepdims=True)
                w = jnp.exp(s - m)
                w = (w / jnp.sum(w, axis=1, keepdims=True)).astype(jnp.bfloat16)
                c = (lax.dot(w[:, :SKV_LOC], v0[:, hs],
                             preferred_element_type=jnp.float32)
                     + lax.dot(w[:, SKV_LOC:], v1[:, hs],
                               preferred_element_type=jnp.float32))
                ctx_heads.append(c)
            ctx_b = jnp.concatenate(ctx_heads, axis=1).astype(jnp.bfloat16)
            pout_b = lax.dot(ctx_b, wob, preferred_element_type=jnp.float32)
            out_ref[b] = pout_b
            pout_send[b] = pout_b.astype(jnp.bfloat16)

        def _p_desc(slot, tgt):
            return pltpu.make_async_remote_copy(
                src_ref=pout_send, dst_ref=pout_recv.at[slot],
                send_sem=p_send_sems.at[slot], recv_sem=p_recv_sems.at[slot],
                device_id=(tgt,), device_id_type=_DeviceIdType.MESH)

        for d in range(1, N_DEV):
            _p_desc(d - 1, (my + d) % N_DEV).start()
        for r in range(N_DEV - 1):
            _p_desc(r, my).wait_recv()

        for b in range(B):
            out_ref[b] = (out_ref[b]
                          + pout_recv[0, b].astype(jnp.float32)
                          + pout_recv[1, b].astype(jnp.float32)
                          + pout_recv[2, b].astype(jnp.float32))

        for d in range(1, N_DEV):
            _p_desc(d - 1, (my + d) % N_DEV).wait_send()
        for p in range(N_SRC):
            @pl.when(my == p)
            def _(p=p):
                for t in range(N_DEV):
                    if t != p:
                        _kv_send_desc(p, t).wait_send()

        @functools.partial(pl.run_scoped, exit_sem=pltpu.SemaphoreType.REGULAR)
        def _(exit_sem):
            for d in range(1, N_DEV):
                _sem_signal(exit_sem, inc=1, device_id=((my + d) % N_DEV,),
                            device_id_type=_DeviceIdType.MESH)
            _sem_wait(exit_sem, N_DEV - 1)

    return pl.pallas_call(
        body,
        out_shape=jax.ShapeDtypeStruct((B, SQ, D_MODEL), jnp.float32),
        in_specs=[pl.BlockSpec(memory_space=pltpu.VMEM)] * 5,
        out_specs=pl.BlockSpec(memory_space=pltpu.VMEM),
        scratch_shapes=[
            pltpu.VMEM((N_DEV, 2, B, SKV_LOC, H_LOC * DH), jnp.bfloat16),
            pltpu.VMEM((N_SRC, 2, B, SKV_LOC, H_LOC * DH), jnp.bfloat16),
            pltpu.VMEM((B, SQ, D_MODEL), jnp.bfloat16),
            pltpu.VMEM((N_DEV - 1, B, SQ, D_MODEL), jnp.bfloat16),
            pltpu.SemaphoreType.DMA((N_DEV,)),
            pltpu.SemaphoreType.DMA((N_SRC,)),
            pltpu.SemaphoreType.DMA((N_DEV - 1,)),
            pltpu.SemaphoreType.DMA((N_DEV - 1,)),
        ],
        compiler_params=_CompilerParams(collective_id=0),
    )(x, Wq, K2, V2, Wo)


# device time: 23298 ns/iter; 1.4787x vs baseline; 1.1729x over previous
import functools

import jax
import jax.numpy as jnp
from jax import lax
from jax.experimental import pallas as pl
from jax.experimental.pallas import tpu as pltpu

N_DEV = 4
N_SRC = 2
B = 2
SQ = 128
SKV_LOC = 128
H_LOC = 4
DH = 64
D_MODEL = 512
NEG = -1e9

_DeviceIdType = getattr(pl, "DeviceIdType", None) or pltpu.DeviceIdType
_sem_signal = getattr(pl, "semaphore_signal", None) or pltpu.semaphore_signal
_sem_wait = getattr(pl, "semaphore_wait", None) or pltpu.semaphore_wait
_CompilerParams = getattr(pltpu, "CompilerParams", None) or getattr(
    pltpu, "TPUCompilerParams"
)


def kernel(x, Wq, K_ext, V_ext, Wo):
    K2 = K_ext.reshape(B, SKV_LOC, 16 * DH)
    V2 = V_ext.reshape(B, SKV_LOC, 16 * DH)
    HD = H_LOC * DH

    def body(x_ref, wq_ref, k_ref, v_ref, wo_ref, out_ref,
             kv_send, kv_recv, pout_send, pout_recv,
             kv_send_sems, kv_recv_sems, p_send_sems, p_recv_sems):
        my = lax.axis_index("i")

        barrier = pltpu.get_barrier_semaphore()
        for d in range(1, N_DEV):
            _sem_signal(barrier, inc=1, device_id=((my + d) % N_DEV,),
                        device_id_type=_DeviceIdType.MESH)
        _sem_wait(barrier, N_DEV - 1)

        def _kv_send_desc(p, t):
            return pltpu.make_async_remote_copy(
                src_ref=kv_send.at[t], dst_ref=kv_recv.at[p],
                send_sem=kv_send_sems.at[t], recv_sem=kv_recv_sems.at[p],
                device_id=(t,), device_id_type=_DeviceIdType.MESH)

        for p in range(N_SRC):
            @pl.when(my == p)
            def _(p=p):
                for t in range(N_DEV):
                    if t == p:
                        continue
                    kv_send[t, 0] = k_ref[:, :, HD * t:HD * (t + 1)].astype(jnp.bfloat16)
                    kv_send[t, 1] = v_ref[:, :, HD * t:HD * (t + 1)].astype(jnp.bfloat16)
                    _kv_send_desc(p, t).start()
                kv_recv[p, 0] = k_ref[:, :, HD * p:HD * (p + 1)].astype(jnp.bfloat16)
                kv_recv[p, 1] = v_ref[:, :, HD * p:HD * (p + 1)].astype(jnp.bfloat16)

        wqb = wq_ref[...].astype(jnp.bfloat16)
        wob = wo_ref[...].astype(jnp.bfloat16)
        qs = []
        for b in range(B):
            qs.append(lax.dot(x_ref[b].astype(jnp.bfloat16), wqb,
                              preferred_element_type=jnp.float32))

        iota_i = lax.broadcasted_iota(jnp.int32, (SQ, SKV_LOC), 0)
        iota_j = lax.broadcasted_iota(jnp.int32, (SQ, SKV_LOC), 1)
        causal = iota_j <= iota_i

        @pl.when(my != 0)
        def _():
            _kv_send_desc(0, 0).wait_recv()

        qhs = [[qs[b][:, h * DH:(h + 1) * DH].astype(jnp.bfloat16)
                for h in range(H_LOC)] for b in range(B)]
        s0s = [[lax.dot_general(qhs[b][h],
                                kv_recv[0, 0, b][:, h * DH:(h + 1) * DH],
                                (((1,), (1,)), ((), ())),
                                preferred_element_type=jnp.float32) * 0.125
                for h in range(H_LOC)] for b in range(B)]

        @pl.when(my != 1)
        def _():
            _kv_send_desc(1, 1).wait_recv()

        def _p_desc(slot, b, tgt):
            return pltpu.make_async_remote_copy(
                src_ref=pout_send.at[b], dst_ref=pout_recv.at[slot, b],
                send_sem=p_send_sems.at[slot, b], recv_sem=p_recv_sems.at[slot, b],
                device_id=(tgt,), device_id_type=_DeviceIdType.MESH)

        for b in range(B):
            k1 = kv_recv[1, 0, b]
            v0 = kv_recv[0, 1, b]
            v1 = kv_recv[1, 1, b]
            ctx_heads = []
            for h in range(H_LOC):
                hs = slice(h * DH, (h + 1) * DH)
                s1 = lax.dot_general(qhs[b][h], k1[:, hs], (((1,), (1,)), ((), ())),
                                     preferred_element_type=jnp.float32) * 0.125
                s1 = jnp.where(causal, s1, NEG)
                s = jnp.concatenate([s0s[b][h], s1], axis=1)
                m = jnp.max(s, axis=1, keepdims=True)
                w = jnp.exp(s - m)
                w = (w / jnp.sum(w, axis=1, keepdims=True)).astype(jnp.bfloat16)
                c = (lax.dot(w[:, :SKV_LOC], v0[:, hs],
                             preferred_element_type=jnp.float32)
                     + lax.dot(w[:, SKV_LOC:], v1[:, hs],
                               preferred_element_type=jnp.float32))
                ctx_heads.append(c)
            ctx_b = jnp.concatenate(ctx_heads, axis=1).astype(jnp.bfloat16)
            pout_b = lax.dot(ctx_b, wob, preferred_element_type=jnp.float32)
            out_ref[b] = pout_b
            pout_send[b] = pout_b.astype(jnp.bfloat16)
            for d in range(1, N_DEV):
                _p_desc(d - 1, b, (my + d) % N_DEV).start()

        for b in range(B):
            for r in range(N_DEV - 1):
                _p_desc(r, b, my).wait_recv()
            out_ref[b] = (out_ref[b]
                          + pout_recv[0, b].astype(jnp.float32)
                          + pout_recv[1, b].astype(jnp.float32)
                          + pout_recv[2, b].astype(jnp.float32))

        for b in range(B):
            for d in range(1, N_DEV):
                _p_desc(d - 1, b, (my + d) % N_DEV).wait_send()
        for p in range(N_SRC):
            @pl.when(my == p)
            def _(p=p):
                for t in range(N_DEV):
                    if t != p:
                        _kv_send_desc(p, t).wait_send()

        @functools.partial(pl.run_scoped, exit_sem=pltpu.SemaphoreType.REGULAR)
        def _(exit_sem):
            for d in range(1, N_DEV):
                _sem_signal(exit_sem, inc=1, device_id=((my + d) % N_DEV,),
                            device_id_type=_DeviceIdType.MESH)
            _sem_wait(exit_sem, N_DEV - 1)

    return pl.pallas_call(
        body,
        out_shape=jax.ShapeDtypeStruct((B, SQ, D_MODEL), jnp.float32),
        in_specs=[pl.BlockSpec(memory_space=pltpu.VMEM)] * 5,
        out_specs=pl.BlockSpec(memory_space=pltpu.VMEM),
        scratch_shapes=[
            pltpu.VMEM((N_DEV, 2, B, SKV_LOC, H_LOC * DH), jnp.bfloat16),
            pltpu.VMEM((N_SRC, 2, B, SKV_LOC, H_LOC * DH), jnp.bfloat16),
            pltpu.VMEM((B, SQ, D_MODEL), jnp.bfloat16),
            pltpu.VMEM((N_DEV - 1, B, SQ, D_MODEL), jnp.bfloat16),
            pltpu.SemaphoreType.DMA((N_DEV,)),
            pltpu.SemaphoreType.DMA((N_SRC,)),
            pltpu.SemaphoreType.DMA((N_DEV - 1, B)),
            pltpu.SemaphoreType.DMA((N_DEV - 1, B)),
        ],
        compiler_params=_CompilerParams(collective_id=0),
    )(x, Wq, K2, V2, Wo)
